# baseline (speedup 1.0000x reference)
import jax
import jax.numpy as jnp
from jax.experimental import pallas as pl
from jax.experimental.pallas import tpu as pltpu

BLOCK_P = 256


def _router_block(h2_ref, w_ref, out_ref):
    w = w_ref[...]
    h2 = h2_ref[...]
    hidden = w.shape[1]

    def probs(h):
        logits = jax.lax.dot_general(
            h, w, (((1,), (1,)), ((), ())), preferred_element_type=jnp.float32
        )
        m = jnp.max(logits, axis=-1, keepdims=True)
        e = jnp.exp(logits - m)
        return e / jnp.sum(e, axis=-1, keepdims=True)

    p_even = probs(h2[:, :hidden])
    p_odd = probs(h2[:, hidden:])
    out_ref[...] = jnp.concatenate([p_even, p_odd], axis=1)


def kernel(hidden_states, gate_weight):
    n_tokens, hidden = hidden_states.shape
    n_experts = gate_weight.shape[0]
    n_pairs = n_tokens // 2
    h2 = hidden_states.reshape(n_pairs, 2 * hidden)
    grid = (n_pairs // BLOCK_P,)
    out = pl.pallas_call(
        _router_block,
        grid=grid,
        in_specs=[
            pl.BlockSpec((BLOCK_P, 2 * hidden), lambda i: (i, 0)),
            pl.BlockSpec((n_experts, hidden), lambda i: (0, 0)),
        ],
        out_specs=pl.BlockSpec((BLOCK_P, 2 * n_experts), lambda i: (i, 0)),
        out_shape=jax.ShapeDtypeStruct((n_pairs, 2 * n_experts), jnp.float32),
        compiler_params=pltpu.CompilerParams(
            dimension_semantics=("arbitrary",),
        ),
    )(h2, gate_weight)
    return out.reshape(n_tokens, n_experts)


# speedup vs baseline: 3.9488x; 3.9488x over previous
import jax
import jax.numpy as jnp
from jax.experimental import pallas as pl
from jax.experimental.pallas import tpu as pltpu

CHUNK = 256
NBUF = 8
LOOK = 6


def _router_kernel(h_hbm, w_ref, out_ref, h_vmem, sems):
    n_tokens = out_ref.shape[0]
    n_chunks = n_tokens // CHUNK
    w = w_ref[...]

    def start_copy(c):
        slot = jax.lax.rem(c, NBUF)
        pltpu.make_async_copy(
            h_hbm.at[pl.ds(c * CHUNK, CHUNK), :],
            h_vmem.at[slot],
            sems.at[slot],
        ).start()

    def wait_copy(c):
        slot = jax.lax.rem(c, NBUF)
        pltpu.make_async_copy(
            h_hbm.at[pl.ds(c * CHUNK, CHUNK), :],
            h_vmem.at[slot],
            sems.at[slot],
        ).wait()

    for j in range(LOOK):
        start_copy(jnp.int32(j))

    def body(c, carry):
        @pl.when(c + LOOK < n_chunks)
        def _():
            start_copy(c + LOOK)

        wait_copy(c)
        slot = jax.lax.rem(c, NBUF)
        h = h_vmem[slot]
        logits = jax.lax.dot_general(
            h, w, (((1,), (1,)), ((), ())), preferred_element_type=jnp.float32
        )
        m = jnp.max(logits, axis=-1, keepdims=True)
        e = jnp.exp(logits - m)
        out_ref[pl.ds(c * CHUNK, CHUNK), :] = e / jnp.sum(e, axis=-1, keepdims=True)
        return carry

    jax.lax.fori_loop(0, n_chunks, body, 0)


def kernel(hidden_states, gate_weight):
    n_tokens, hidden = hidden_states.shape
    n_experts = gate_weight.shape[0]
    return pl.pallas_call(
        _router_kernel,
        in_specs=[
            pl.BlockSpec(memory_space=pltpu.MemorySpace.HBM),
            pl.BlockSpec((n_experts, hidden), lambda: (0, 0)),
        ],
        out_specs=pl.BlockSpec((n_tokens, n_experts), lambda: (0, 0)),
        out_shape=jax.ShapeDtypeStruct((n_tokens, n_experts), jnp.float32),
        scratch_shapes=[
            pltpu.VMEM((NBUF, CHUNK, hidden), jnp.float32),
            pltpu.SemaphoreType.DMA((NBUF,)),
        ],
    )(hidden_states, gate_weight)


# speedup vs baseline: 4.0777x; 1.0327x over previous
import jax
import jax.numpy as jnp
from jax.experimental import pallas as pl
from jax.experimental.pallas import tpu as pltpu

BLOCK_M = 512


def _router_block(h_top_ref, h_bot_ref, w_ref, out_ref):
    w = w_ref[...]

    def probs(h):
        logits = jax.lax.dot_general(
            h, w, (((1,), (1,)), ((), ())), preferred_element_type=jnp.float32
        )
        m = jnp.max(logits, axis=-1, keepdims=True)
        e = jnp.exp(logits - m)
        return e / jnp.sum(e, axis=-1, keepdims=True)

    out_ref[0] = probs(h_top_ref[...])
    out_ref[1] = probs(h_bot_ref[...])


def kernel(hidden_states, gate_weight):
    n_tokens, hidden = hidden_states.shape
    n_experts = gate_weight.shape[0]
    half_blocks = n_tokens // (2 * BLOCK_M)
    grid = (half_blocks,)
    out = pl.pallas_call(
        _router_block,
        grid=grid,
        in_specs=[
            pl.BlockSpec((BLOCK_M, hidden), lambda i: (i, 0)),
            pl.BlockSpec((BLOCK_M, hidden), lambda i, nb=half_blocks: (i + nb, 0)),
            pl.BlockSpec((n_experts, hidden), lambda i: (0, 0)),
        ],
        out_specs=pl.BlockSpec((2, BLOCK_M, n_experts), lambda i: (0, i, 0)),
        out_shape=jax.ShapeDtypeStruct((2, n_tokens // 2, n_experts), jnp.float32),
        compiler_params=pltpu.CompilerParams(
            dimension_semantics=("arbitrary",),
        ),
    )(hidden_states, hidden_states, gate_weight)
    return out.reshape(n_tokens, n_experts)
